# baseline (speedup 1.0000x reference)
import jax
import jax.numpy as jnp
from jax import lax
from jax.experimental import pallas as pl
from jax.experimental.pallas import tpu as pltpu
from jax.experimental.pallas import tpu_sc as plsc

BATCH = 128
SEQ = 650
HIDDEN = 1024
TPF = 65
NFRAME = 10

try:
    _INFO = plsc.get_sparse_core_info()
    NC = _INFO.num_cores
    NS = _INFO.num_subcores
except ValueError:
    NC, NS = 2, 16
NW = NC * NS

G = 16
NV = (SEQ + G - 1) // G
NTASK = NV * BATCH
TPW = NTASK // NW
NVREG = HIDDEN // 16
NBUF = 3
NFULL = (TPW // NBUF) * NBUF


def _bias_add(buf, bias_flat, lv):
    def row_body(r, _):
        base = (lv * G + r) * HIDDEN
        for c in range(NVREG):
            plsc.addupdate(buf.at[r, pl.ds(c * 16, 16)],
                           bias_flat[pl.ds(base + c * 16, 16)])
        return 0

    lax.fori_loop(0, G, row_body, 0)


def _sc_body(tok_idx_hbm, out_idx_hbm, bias_hbm, table_hbm, out_hbm,
             tok_idx, out_idx, bias, bufs, gsems, ssems):
    w = lax.axis_index("s") * NC + lax.axis_index("c")
    t0 = w * TPW
    first = t0 // BATCH
    pltpu.sync_copy(tok_idx_hbm.at[pl.ds(t0 * G, TPW * G)], tok_idx)
    pltpu.sync_copy(out_idx_hbm.at[pl.ds(t0 * G, TPW * G)], out_idx)
    pltpu.sync_copy(bias_hbm.at[pl.ds(first * G * HIDDEN, 3 * G * HIDDEN)],
                    bias)

    def start_gather(ti, k):
        pltpu.async_copy(table_hbm.at[tok_idx.at[pl.ds(ti * G, G)]],
                         bufs[k], gsems[k])

    def wait_gather(ti, k):
        pltpu.make_async_copy(table_hbm.at[tok_idx.at[pl.ds(ti * G, G)]],
                              bufs[k], gsems[k]).wait()

    def start_scatter(ti, k):
        pltpu.async_copy(bufs[k], out_hbm.at[out_idx.at[pl.ds(ti * G, G)]],
                         ssems[k])

    def wait_scatter(ti, k):
        pltpu.make_async_copy(bufs[k], out_hbm.at[out_idx.at[pl.ds(ti * G, G)]],
                              ssems[k]).wait()

    def process(ti, k):
        lv = (t0 + ti) // BATCH - first
        wait_gather(ti, k)
        _bias_add(bufs[k], bias, lv)
        start_scatter(ti, k)

    for k in range(NBUF):
        start_gather(k, k)

    def super_body(g, _):
        for k in range(NBUF):
            process(g * NBUF + k, k)
        for k in range(NBUF):
            ti = g * NBUF + k

            @pl.when(ti + NBUF < TPW)
            def _():
                wait_scatter(ti, k)
                start_gather(ti + NBUF, k)

        return 0

    lax.fori_loop(0, TPW // NBUF, super_body, 0)

    for ti in range(NFULL, TPW):
        process(ti, ti % NBUF)

    for ti in range(TPW - NBUF, TPW):
        wait_scatter(ti, ti % NBUF)


def _make_kernel():
    mesh = plsc.VectorSubcoreMesh(core_axis_name="c", subcore_axis_name="s")
    return pl.kernel(
        _sc_body,
        out_type=jax.ShapeDtypeStruct((BATCH * SEQ, HIDDEN), jnp.float32),
        mesh=mesh,
        scratch_types=[
            pltpu.VMEM((TPW * G,), jnp.int32),
            pltpu.VMEM((TPW * G,), jnp.int32),
            pltpu.VMEM((3 * G * HIDDEN,), jnp.float32),
            [pltpu.VMEM((G, HIDDEN), jnp.float32) for _ in range(NBUF)],
            [pltpu.SemaphoreType.DMA for _ in range(NBUF)],
            [pltpu.SemaphoreType.DMA for _ in range(NBUF)],
        ],
    )


@jax.jit
def kernel(tokens, token_table, pos_table, frame_table):
    tokens = tokens.astype(jnp.int32)
    wstarts = [min(v * G, SEQ - G) for v in range(NV)]
    tok_idx = jnp.stack([tokens[:, s:s + G] for s in wstarts])
    sv = jnp.asarray(wstarts, dtype=jnp.int32)
    out_idx = (jnp.arange(BATCH, dtype=jnp.int32)[None, :, None] * SEQ
               + sv[:, None, None]
               + jnp.arange(G, dtype=jnp.int32)[None, None, :])
    combined = (jnp.tile(pos_table, (NFRAME, 1))
                + jnp.repeat(frame_table, TPF, axis=0))
    bias_all = jnp.stack([combined[s:s + G] for s in wstarts]
                         + [combined[wstarts[-1]:wstarts[-1] + G]])
    out_flat = _make_kernel()(tok_idx.reshape(-1), out_idx.reshape(-1),
                              bias_all.reshape(-1), token_table)
    return out_flat.reshape(BATCH, SEQ, HIDDEN)


# speedup vs baseline: 1.0177x; 1.0177x over previous
import jax
import jax.numpy as jnp
from jax import lax
from jax.experimental import pallas as pl
from jax.experimental.pallas import tpu as pltpu
from jax.experimental.pallas import tpu_sc as plsc

BATCH = 128
SEQ = 650
HIDDEN = 1024
TPF = 65
NFRAME = 10

try:
    _INFO = plsc.get_sparse_core_info()
    NC = _INFO.num_cores
    NS = _INFO.num_subcores
except ValueError:
    NC, NS = 2, 16
NW = NC * NS

G = 16
NV = (SEQ + G - 1) // G
NTASK = NV * BATCH
TPW = NTASK // NW
NVREG = HIDDEN // 16
NBUF = 2
NPAIR = TPW // 2


def _bias_add(buf, bias_flat, lv):
    def row_body(r, _):
        base = (lv * G + r) * HIDDEN
        for c in range(NVREG):
            plsc.addupdate(buf.at[r, pl.ds(c * 16, 16)],
                           bias_flat[pl.ds(base + c * 16, 16)])
        return 0

    lax.fori_loop(0, G, row_body, 0)


def _sc_body(tok_idx_hbm, out_idx_hbm, bias_hbm, table_hbm, out_hbm,
             tok_idx, out_idx, bias, bufs, gsems, ssems):
    w = lax.axis_index("s") * NC + lax.axis_index("c")
    t0 = w * TPW
    first = t0 // BATCH
    pltpu.sync_copy(tok_idx_hbm.at[pl.ds(t0 * G, TPW * G)], tok_idx)
    pltpu.sync_copy(out_idx_hbm.at[pl.ds(t0 * G, TPW * G)], out_idx)
    pltpu.sync_copy(bias_hbm.at[pl.ds(first * G * HIDDEN, 3 * G * HIDDEN)],
                    bias)

    def start_gather(p, k):
        pltpu.async_copy(table_hbm.at[tok_idx.at[pl.ds(p * 2 * G, 2 * G)]],
                         bufs[k], gsems[k])

    def wait_gather(p, k):
        pltpu.make_async_copy(table_hbm.at[tok_idx.at[pl.ds(p * 2 * G, 2 * G)]],
                              bufs[k], gsems[k]).wait()

    def start_scatter(p, k):
        pltpu.async_copy(bufs[k], out_hbm.at[out_idx.at[pl.ds(p * 2 * G, 2 * G)]],
                         ssems[k])

    def wait_scatter(p, k):
        pltpu.make_async_copy(bufs[k],
                              out_hbm.at[out_idx.at[pl.ds(p * 2 * G, 2 * G)]],
                              ssems[k]).wait()

    def process(p, k):
        lv0 = (t0 + 2 * p) // BATCH - first
        lv1 = (t0 + 2 * p + 1) // BATCH - first
        wait_gather(p, k)
        _bias_add(bufs[k].at[pl.ds(0, G)], bias, lv0)
        _bias_add(bufs[k].at[pl.ds(G, G)], bias, lv1)
        start_scatter(p, k)

    for k in range(NBUF):
        start_gather(k, k)

    def super_body(g, _):
        for k in range(NBUF):
            process(g * NBUF + k, k)
        for k in range(NBUF):
            p = g * NBUF + k

            @pl.when(p + NBUF < NPAIR)
            def _():
                wait_scatter(p, k)
                start_gather(p + NBUF, k)

        return 0

    lax.fori_loop(0, NPAIR // NBUF, super_body, 0)

    for p in range(NPAIR - NBUF, NPAIR):
        wait_scatter(p, p % NBUF)


def _make_kernel():
    mesh = plsc.VectorSubcoreMesh(core_axis_name="c", subcore_axis_name="s")
    return pl.kernel(
        _sc_body,
        out_type=jax.ShapeDtypeStruct((BATCH * SEQ, HIDDEN), jnp.float32),
        mesh=mesh,
        scratch_types=[
            pltpu.VMEM((TPW * G,), jnp.int32),
            pltpu.VMEM((TPW * G,), jnp.int32),
            pltpu.VMEM((3 * G * HIDDEN,), jnp.float32),
            [pltpu.VMEM((2 * G, HIDDEN), jnp.float32) for _ in range(NBUF)],
            [pltpu.SemaphoreType.DMA for _ in range(NBUF)],
            [pltpu.SemaphoreType.DMA for _ in range(NBUF)],
        ],
    )


@jax.jit
def kernel(tokens, token_table, pos_table, frame_table):
    tokens = tokens.astype(jnp.int32)
    wstarts = [min(v * G, SEQ - G) for v in range(NV)]
    tok_idx = jnp.stack([tokens[:, s:s + G] for s in wstarts])
    sv = jnp.asarray(wstarts, dtype=jnp.int32)
    out_idx = (jnp.arange(BATCH, dtype=jnp.int32)[None, :, None] * SEQ
               + sv[:, None, None]
               + jnp.arange(G, dtype=jnp.int32)[None, None, :])
    combined = (jnp.tile(pos_table, (NFRAME, 1))
                + jnp.repeat(frame_table, TPF, axis=0))
    bias_all = jnp.stack([combined[s:s + G] for s in wstarts]
                         + [combined[wstarts[-1]:wstarts[-1] + G]])
    out_flat = _make_kernel()(tok_idx.reshape(-1), out_idx.reshape(-1),
                              bias_all.reshape(-1), token_table)
    return out_flat.reshape(BATCH, SEQ, HIDDEN)


# speedup vs baseline: 1.6216x; 1.5933x over previous
import jax
import jax.numpy as jnp
from jax import lax
from jax.experimental import pallas as pl
from jax.experimental.pallas import tpu as pltpu
from jax.experimental.pallas import tpu_sc as plsc

BATCH = 128
SEQ = 650
HIDDEN = 1024
TPF = 65
NFRAME = 10
SEQP = 656

try:
    _INFO = plsc.get_sparse_core_info()
    NC = _INFO.num_cores
    NS = _INFO.num_subcores
except ValueError:
    NC, NS = 2, 16
NW = NC * NS

G = 16
NV = (SEQ + G - 1) // G
NTASK = NV * BATCH
TPW = NTASK // NW
NVREG = HIDDEN // 16
NBUF = 2
NPAIR = TPW // 2


def _bias_add(buf, bias_flat, lv):
    def row_body(r, _):
        base = (lv * G + r) * HIDDEN
        for c in range(NVREG):
            plsc.addupdate(buf.at[r, pl.ds(c * 16, 16)],
                           bias_flat[pl.ds(base + c * 16, 16)])
        return 0

    lax.fori_loop(0, G, row_body, 0)


def _sc_body(tok_idx_hbm, out_idx_hbm, table_hbm, out_hbm,
             tok_idx, out_idx, bufs, gsems, ssems):
    w = lax.axis_index("s") * NC + lax.axis_index("c")
    t0 = w * TPW
    first = t0 // BATCH
    pltpu.sync_copy(tok_idx_hbm.at[pl.ds(t0 * G, TPW * G)], tok_idx)
    pltpu.sync_copy(out_idx_hbm.at[pl.ds(t0 * G, TPW * G)], out_idx)

    def start_gather(p, k):
        pltpu.async_copy(table_hbm.at[tok_idx.at[pl.ds(p * 2 * G, 2 * G)]],
                         bufs[k], gsems[k])

    def wait_gather(p, k):
        pltpu.make_async_copy(table_hbm.at[tok_idx.at[pl.ds(p * 2 * G, 2 * G)]],
                              bufs[k], gsems[k]).wait()

    def start_scatter(p, k):
        pltpu.async_copy(bufs[k], out_hbm.at[out_idx.at[pl.ds(p * 2 * G, 2 * G)]],
                         ssems[k])

    def wait_scatter(p, k):
        pltpu.make_async_copy(bufs[k],
                              out_hbm.at[out_idx.at[pl.ds(p * 2 * G, 2 * G)]],
                              ssems[k]).wait()

    def process(p, k):
        wait_gather(p, k)
        start_scatter(p, k)

    for k in range(NBUF):
        start_gather(k, k)

    def super_body(g, _):
        for k in range(NBUF):
            process(g * NBUF + k, k)
        for k in range(NBUF):
            p = g * NBUF + k

            @pl.when(p + NBUF < NPAIR)
            def _():
                wait_scatter(p, k)
                start_gather(p + NBUF, k)

        return 0

    lax.fori_loop(0, NPAIR // NBUF, super_body, 0)

    for p in range(NPAIR - NBUF, NPAIR):
        wait_scatter(p, p % NBUF)


def _make_kernel():
    mesh = plsc.VectorSubcoreMesh(core_axis_name="c", subcore_axis_name="s")
    return pl.kernel(
        _sc_body,
        out_type=jax.ShapeDtypeStruct((BATCH * SEQP, HIDDEN), jnp.float32),
        mesh=mesh,
        scratch_types=[
            pltpu.VMEM((TPW * G,), jnp.int32),
            pltpu.VMEM((TPW * G,), jnp.int32),
            [pltpu.VMEM((2 * G, HIDDEN), jnp.float32) for _ in range(NBUF)],
            [pltpu.SemaphoreType.DMA for _ in range(NBUF)],
            [pltpu.SemaphoreType.DMA for _ in range(NBUF)],
        ],
    )


def _tc_add_body(flat_ref, bias_ref, out_ref):
    out_ref[...] = (flat_ref[...] + bias_ref[...])[None, :SEQ]


def _make_tc_add():
    return pl.pallas_call(
        _tc_add_body,
        grid=(BATCH,),
        in_specs=[
            pl.BlockSpec((SEQP, HIDDEN), lambda b: (b, 0)),
            pl.BlockSpec((SEQP, HIDDEN), lambda b: (0, 0)),
        ],
        out_specs=pl.BlockSpec((1, SEQ, HIDDEN), lambda b: (b, 0, 0)),
        out_shape=jax.ShapeDtypeStruct((BATCH, SEQ, HIDDEN), jnp.float32),
    )


@jax.jit
def kernel(tokens, token_table, pos_table, frame_table):
    tokens = tokens.astype(jnp.int32)
    wstarts = [min(v * G, SEQ - G) for v in range(NV)]
    tok_idx = jnp.stack([tokens[:, s:s + G] for s in wstarts])
    sv = jnp.asarray(wstarts, dtype=jnp.int32)
    out_idx = (jnp.arange(BATCH, dtype=jnp.int32)[None, :, None] * SEQP
               + sv[:, None, None]
               + jnp.arange(G, dtype=jnp.int32)[None, None, :])
    combined = (jnp.tile(pos_table, (NFRAME, 1))
                + jnp.repeat(frame_table, TPF, axis=0))
    combined = jnp.concatenate(
        [combined, jnp.zeros((SEQP - SEQ, HIDDEN), jnp.float32)])
    gat_flat = _make_kernel()(tok_idx.reshape(-1), out_idx.reshape(-1),
                              token_table)
    return _make_tc_add()(gat_flat, combined)
